# baseline (speedup 1.0000x reference)
import functools

import jax
import jax.numpy as jnp
from jax import lax
from jax.experimental import pallas as pl
from jax.experimental.pallas import tpu as pltpu
from jax.experimental.pallas import tpu_sc as plsc

NUM_CORES = 2
NUM_SUBCORES = 16
NUM_WORKERS = NUM_CORES * NUM_SUBCORES
NBUF = 4
L = 16


@functools.lru_cache(maxsize=None)
def _make_widen(dim: int, vocab: int):
    grid = (vocab + 127) // 128

    def body(wt_ref, out_ref):
        xt = wt_ref[...].T
        out_ref[...] = jnp.concatenate([xt, xt], axis=1)

    return pl.pallas_call(
        body,
        grid=(grid,),
        in_specs=[pl.BlockSpec((dim, 128), lambda i: (0, i))],
        out_specs=pl.BlockSpec((128, 2 * dim), lambda i: (i, 0)),
        out_shape=jax.ShapeDtypeStruct((vocab, 2 * dim), jnp.float32),
    )


@functools.lru_cache(maxsize=None)
def _make_gather(batch: int, seq: int, dim: int):
    assert batch % NUM_WORKERS == 0
    b_per_w = batch // NUM_WORKERS
    assert b_per_w == 128 and dim == 64 and seq % NBUF == 0
    mesh = plsc.VectorSubcoreMesh(core_axis_name="c", subcore_axis_name="s")

    def body(idxT_hbm, wide_hbm, out_hbm, idx_v, gbufs, tbufs, *sems):
        gsem = sems[:NBUF]
        wsem = sems[NBUF:]
        wid = lax.axis_index("s") * NUM_CORES + lax.axis_index("c")
        b0 = wid * b_per_w
        pltpu.sync_copy(idxT_hbm.at[:, pl.ds(b0, b_per_w)], idx_v)
        lanes = lax.iota(jnp.int32, L)

        def issue_gather(s, k):
            pltpu.async_copy(wide_hbm.at[idx_v.at[s]], gbufs.at[k], gsem[k])

        def wait_gather(k):
            pltpu.make_async_copy(
                wide_hbm.at[idx_v.at[0]], gbufs.at[k], gsem[k]).wait()

        def issue_write(s, k):
            pltpu.async_copy(tbufs.at[k], out_hbm.at[s, :, wid], wsem[k])

        def wait_write(k):
            pltpu.make_async_copy(
                tbufs.at[k], out_hbm.at[0, :, wid], wsem[k]).wait()

        for s in range(NBUF):
            issue_gather(s, s)

        @pl.loop(0, seq, step=NBUF)
        def _(s0):
            for k in range(NBUF):
                s = s0 + k
                wait_gather(k)

                @pl.when(s >= NBUF)
                def _():
                    wait_write(k)

                for dt in range(8):
                    @pl.loop(0, 8)
                    def _(r):
                        d = jnp.int32(dt * 8) + r
                        for j in range(8):
                            vals = plsc.load_gather(
                                gbufs.at[k],
                                [j * L + lanes,
                                 jnp.broadcast_to(d, (L,))])
                            tbufs[k, dt, r, pl.ds(j * L, L)] = vals

                issue_write(s, k)

                @pl.when(s + NBUF < seq)
                def _():
                    issue_gather(s + NBUF, k)

        for k in range(NBUF):
            wait_write(k)

    return pl.kernel(
        body,
        out_type=jax.ShapeDtypeStruct((seq, 8, NUM_WORKERS, 8, 128),
                                      jnp.float32),
        mesh=mesh,
        scratch_types=[
            pltpu.VMEM((seq, b_per_w), jnp.int32),
            pltpu.VMEM((NBUF, b_per_w, 128), jnp.float32),
            pltpu.VMEM((NBUF, 8, 8, 128), jnp.float32),
        ] + [pltpu.SemaphoreType.DMA] * (2 * NBUF),
        compiler_params=pltpu.CompilerParams(
            use_tc_tiling_on_sc=False, needs_layout_passes=False),
    )


def kernel(token_ids, weight):
    batch, seq = token_ids.shape
    vocab, dim = weight.shape
    idxT = token_ids.T.astype(jnp.int32)
    wide = _make_widen(dim, vocab)(weight.T)
    out5 = _make_gather(batch, seq, dim)(idxT, wide)
    return out5.transpose(2, 4, 0, 1, 3).reshape(batch, seq, dim)


# speedup vs baseline: 2.8044x; 2.8044x over previous
import functools

import jax
import jax.numpy as jnp
from jax import lax
from jax.experimental import pallas as pl
from jax.experimental.pallas import tpu as pltpu
from jax.experimental.pallas import tpu_sc as plsc

NUM_CORES = 2
NUM_SUBCORES = 16
NUM_WORKERS = NUM_CORES * NUM_SUBCORES
NBUF = 2
L = 16


@functools.lru_cache(maxsize=None)
def _make_widen(dim: int, vocab: int):
    blk = 2048
    grid = (vocab + blk - 1) // blk

    def body(wt_ref, out_ref):
        xt = wt_ref[...].T
        out_ref[...] = jnp.concatenate([xt, xt], axis=1)

    return pl.pallas_call(
        body,
        grid=(grid,),
        in_specs=[pl.BlockSpec((dim, blk), lambda i: (0, i))],
        out_specs=pl.BlockSpec((blk, 2 * dim), lambda i: (i, 0)),
        out_shape=jax.ShapeDtypeStruct((vocab, 2 * dim), jnp.float32),
    )


@functools.lru_cache(maxsize=None)
def _make_gather(batch: int, seq: int, dim: int):
    assert batch % NUM_WORKERS == 0
    b_per_w = batch // NUM_WORKERS
    assert b_per_w == 128 and dim == 64 and seq % NBUF == 0
    mesh = plsc.VectorSubcoreMesh(core_axis_name="c", subcore_axis_name="s")

    def body(idxT_hbm, wide_hbm, out_hbm, idx_v, gbufs, tbufs, *sems):
        gsem = sems[:NBUF]
        wsem = sems[NBUF:]
        wid = lax.axis_index("s") * NUM_CORES + lax.axis_index("c")
        b0 = wid * b_per_w
        pltpu.sync_copy(idxT_hbm.at[:, pl.ds(b0, b_per_w)], idx_v)
        lanes = lax.iota(jnp.int32, L)

        def issue_gather(s, k):
            pltpu.async_copy(wide_hbm.at[idx_v.at[s]], gbufs.at[k], gsem[k])

        def wait_gather(k):
            pltpu.make_async_copy(
                wide_hbm.at[idx_v.at[0]], gbufs.at[k], gsem[k]).wait()

        def issue_write(s, k):
            pltpu.async_copy(tbufs.at[k], out_hbm.at[s, :, wid], wsem[k])

        def wait_write(k):
            pltpu.make_async_copy(
                tbufs.at[k], out_hbm.at[0, :, wid], wsem[k]).wait()

        for s in range(NBUF):
            issue_gather(s, s)

        @pl.loop(0, seq, step=NBUF)
        def _(s0):
            for k in range(NBUF):
                s = s0 + k
                wait_gather(k)

                @pl.when(s >= NBUF)
                def _():
                    wait_write(k)

                for dt in range(8):
                    for r in range(8):
                        d_vec = jnp.full((L,), dt * 8 + r, jnp.int32)
                        for j in range(8):
                            vals = plsc.load_gather(
                                gbufs.at[k], [j * L + lanes, d_vec])
                            tbufs[k, dt, r, pl.ds(j * L, L)] = vals

                issue_write(s, k)

                @pl.when(s + NBUF < seq)
                def _():
                    issue_gather(s + NBUF, k)

        for k in range(NBUF):
            wait_write(k)

    return pl.kernel(
        body,
        out_type=jax.ShapeDtypeStruct((seq, 8, NUM_WORKERS, 8, 128),
                                      jnp.float32),
        mesh=mesh,
        scratch_types=[
            pltpu.VMEM((seq, b_per_w), jnp.int32),
            pltpu.VMEM((NBUF, b_per_w, 128), jnp.float32),
            pltpu.VMEM((NBUF, 8, 8, 128), jnp.float32),
        ] + [pltpu.SemaphoreType.DMA] * (2 * NBUF),
        compiler_params=pltpu.CompilerParams(
            use_tc_tiling_on_sc=False, needs_layout_passes=False),
    )


def kernel(token_ids, weight):
    batch, seq = token_ids.shape
    vocab, dim = weight.shape
    idxT = token_ids.T.astype(jnp.int32)
    wide = _make_widen(dim, vocab)(weight.T)
    out5 = _make_gather(batch, seq, dim)(idxT, wide)
    return out5.transpose(2, 4, 0, 1, 3).reshape(batch, seq, dim)


# speedup vs baseline: 3.5058x; 1.2501x over previous
import functools

import jax
import jax.numpy as jnp
from jax import lax
from jax.experimental import pallas as pl
from jax.experimental.pallas import tpu as pltpu
from jax.experimental.pallas import tpu_sc as plsc

NUM_CORES = 2
NUM_SUBCORES = 16
NUM_WORKERS = NUM_CORES * NUM_SUBCORES
NBUF = 2
L = 16


@functools.lru_cache(maxsize=None)
def _make_widen(dim: int, vocab: int):
    blk = 4096
    grid = (vocab + blk - 1) // blk

    def body(wt_ref, out_ref):
        out_ref[:, :dim] = wt_ref[...].T

    return pl.pallas_call(
        body,
        grid=(grid,),
        in_specs=[pl.BlockSpec((dim, blk), lambda i: (0, i))],
        out_specs=pl.BlockSpec((blk, 2 * dim), lambda i: (i, 0)),
        out_shape=jax.ShapeDtypeStruct((vocab, 2 * dim), jnp.float32),
    )


@functools.lru_cache(maxsize=None)
def _make_gather(batch: int, seq: int, dim: int):
    assert batch % NUM_WORKERS == 0
    b_per_w = batch // NUM_WORKERS
    assert b_per_w == 128 and dim == 64 and seq % NBUF == 0
    mesh = plsc.VectorSubcoreMesh(core_axis_name="c", subcore_axis_name="s")

    def body(idxT_hbm, wide_hbm, out_hbm, idx_v, gbufs, tbufs, *sems):
        gsem = sems[:NBUF]
        wsem = sems[NBUF:]
        wid = lax.axis_index("s") * NUM_CORES + lax.axis_index("c")
        b0 = wid * b_per_w
        pltpu.sync_copy(idxT_hbm.at[:, pl.ds(b0, b_per_w)], idx_v)
        lanes = lax.iota(jnp.int32, L)

        def issue_gather(s, k):
            pltpu.async_copy(wide_hbm.at[idx_v.at[s]],
                             gbufs.at[pl.ds(k * 128, 128)], gsem[k])

        def wait_gather(k):
            pltpu.make_async_copy(
                wide_hbm.at[idx_v.at[0]],
                gbufs.at[pl.ds(k * 128, 128)], gsem[k]).wait()

        def issue_write(s, k):
            pltpu.async_copy(tbufs.at[k], out_hbm.at[s, :, wid], wsem[k])

        def wait_write(k):
            pltpu.make_async_copy(
                tbufs.at[k], out_hbm.at[0, :, wid], wsem[k]).wait()

        for s in range(NBUF):
            issue_gather(s, s)

        @pl.loop(0, seq, step=NBUF)
        def _(s0):
            for k in range(NBUF):
                s = s0 + k
                wait_gather(k)

                @pl.when(s >= NBUF)
                def _():
                    wait_write(k)

                @pl.loop(0, dim)
                def _(d):
                    dt = d // 8
                    r = d - dt * 8
                    dv = jnp.broadcast_to(d, (L,))
                    for j in range(8):
                        vals = plsc.load_gather(
                            gbufs, [k * 128 + j * L + lanes, dv])
                        tbufs[k, dt, r, pl.ds(j * L, L)] = vals

                issue_write(s, k)

                @pl.when(s + NBUF < seq)
                def _():
                    issue_gather(s + NBUF, k)

        for k in range(NBUF):
            wait_write(k)

    return pl.kernel(
        body,
        out_type=jax.ShapeDtypeStruct((seq, 8, NUM_WORKERS, 8, 128),
                                      jnp.float32),
        mesh=mesh,
        scratch_types=[
            pltpu.VMEM((seq, b_per_w), jnp.int32),
            pltpu.VMEM((NBUF * b_per_w, 128), jnp.float32),
            pltpu.VMEM((NBUF, 8, 8, 128), jnp.float32),
        ] + [pltpu.SemaphoreType.DMA] * (2 * NBUF),
        compiler_params=pltpu.CompilerParams(
            use_tc_tiling_on_sc=False, needs_layout_passes=False),
    )


def kernel(token_ids, weight):
    batch, seq = token_ids.shape
    vocab, dim = weight.shape
    idxT = token_ids.T.astype(jnp.int32)
    wide = _make_widen(dim, vocab)(weight.T)
    out5 = _make_gather(batch, seq, dim)(idxT, wide)
    return out5.transpose(2, 4, 0, 1, 3).reshape(batch, seq, dim)


# speedup vs baseline: 5.5192x; 1.5743x over previous
import functools

import jax
import jax.numpy as jnp
from jax import lax
from jax.experimental import pallas as pl
from jax.experimental.pallas import tpu as pltpu
from jax.experimental.pallas import tpu_sc as plsc

NUM_CORES = 2
NUM_SUBCORES = 16
NUM_WORKERS = NUM_CORES * NUM_SUBCORES
NBUF = 2
L = 16


@functools.lru_cache(maxsize=None)
def _make_widen(dim: int, vocab: int):
    blk = 4096
    grid = (vocab + blk - 1) // blk

    def body(wt_ref, out_ref):
        out_ref[:, :dim] = wt_ref[...].T

    return pl.pallas_call(
        body,
        grid=(grid,),
        in_specs=[pl.BlockSpec((dim, blk), lambda i: (0, i))],
        out_specs=pl.BlockSpec((blk, 2 * dim), lambda i: (i, 0)),
        out_shape=jax.ShapeDtypeStruct((vocab, 2 * dim), jnp.float32),
    )


@functools.lru_cache(maxsize=None)
def _make_gather(batch: int, seq: int, dim: int):
    assert batch % NUM_WORKERS == 0
    b_per_w = batch // NUM_WORKERS
    assert b_per_w == 128 and dim == 64 and seq % NBUF == 0
    mesh = plsc.VectorSubcoreMesh(core_axis_name="c", subcore_axis_name="s")

    def body(idxT_hbm, wide_hbm, out_hbm, idx_v, gbufs, tbufs, *sems):
        gsem = sems[:NBUF]
        wsem = sems[NBUF:]
        wid = lax.axis_index("s") * NUM_CORES + lax.axis_index("c")
        b0 = wid * b_per_w
        pltpu.sync_copy(idxT_hbm.at[:, pl.ds(b0, b_per_w)], idx_v)
        lanes = lax.iota(jnp.int32, L)

        def issue_gather(s, k):
            pltpu.async_copy(wide_hbm.at[idx_v.at[s]],
                             gbufs.at[pl.ds(k * 128, 128)], gsem[k])

        def wait_gather(k):
            pltpu.make_async_copy(
                wide_hbm.at[idx_v.at[0]],
                gbufs.at[pl.ds(k * 128, 128)], gsem[k]).wait()

        def issue_write(s, k):
            pltpu.async_copy(tbufs.at[k], out_hbm.at[s, :, wid], wsem[k])

        def wait_write(k):
            pltpu.make_async_copy(
                tbufs.at[k], out_hbm.at[0, :, wid], wsem[k]).wait()

        for s in range(NBUF):
            issue_gather(s, s)

        @pl.loop(0, seq, step=NBUF)
        def _(s0):
            for k in range(NBUF):
                s = s0 + k
                wait_gather(k)

                @pl.when(s >= NBUF)
                def _():
                    wait_write(k)

                @plsc.parallel_loop(0, dim, unroll=4)
                def _(d):
                    dt = d // 8
                    r = d - dt * 8
                    dv = jnp.broadcast_to(d, (L,))
                    for j in range(8):
                        vals = plsc.load_gather(
                            gbufs, [k * 128 + j * L + lanes, dv])
                        tbufs[k, dt, r, pl.ds(j * L, L)] = vals

                issue_write(s, k)

                @pl.when(s + NBUF < seq)
                def _():
                    issue_gather(s + NBUF, k)

        for k in range(NBUF):
            wait_write(k)

    return pl.kernel(
        body,
        out_type=jax.ShapeDtypeStruct((seq, 8, NUM_WORKERS, 8, 128),
                                      jnp.float32),
        mesh=mesh,
        scratch_types=[
            pltpu.VMEM((seq, b_per_w), jnp.int32),
            pltpu.VMEM((NBUF * b_per_w, 128), jnp.float32),
            pltpu.VMEM((NBUF, 8, 8, 128), jnp.float32),
        ] + [pltpu.SemaphoreType.DMA] * (2 * NBUF),
        compiler_params=pltpu.CompilerParams(
            use_tc_tiling_on_sc=False, needs_layout_passes=False),
    )


def kernel(token_ids, weight):
    batch, seq = token_ids.shape
    vocab, dim = weight.shape
    idxT = token_ids.T.astype(jnp.int32)
    wide = _make_widen(dim, vocab)(weight.T)
    out5 = _make_gather(batch, seq, dim)(idxT, wide)
    return out5.transpose(2, 4, 0, 1, 3).reshape(batch, seq, dim)
